# initial kernel scaffold (unmeasured)
import jax
import jax.numpy as jnp
from jax import lax
from jax.experimental import pallas as pl
from jax.experimental.pallas import tpu as pltpu

N_DEV = 4
B, SQ, D = 2, 128, 512
HQ, DH = 4, 64
DQ = HQ * DH
ROWS = B * SQ


def kernel(x, Wq, Wk, Wv, Wo):
    def body(x_ref, wq_ref, wk_ref, wv_ref, wo_ref, out_ref,
             comm_ref, send_sems, recv_sems):
        my = lax.axis_index("i")
        left = lax.rem(my + N_DEV - 1, N_DEV)
        right = lax.rem(my + 1, N_DEV)

        barrier_sem = pltpu.get_barrier_semaphore()
        for nbr in (left, right):
            pl.semaphore_signal(
                barrier_sem, inc=1,
                device_id=(nbr,), device_id_type=pl.DeviceIdType.MESH,
            )
        pl.semaphore_wait(barrier_sem, 2)

        xf = x_ref[...].reshape(ROWS, D).astype(jnp.bfloat16)
        wq = wq_ref[...].astype(jnp.bfloat16)
        wk = wk_ref[...].astype(jnp.bfloat16)
        wv = wv_ref[...].astype(jnp.bfloat16)

        q = jnp.dot(xf, wq, preferred_element_type=jnp.float32)
        k = jnp.dot(xf, wk, preferred_element_type=jnp.float32)
        v = jnp.dot(xf, wv, preferred_element_type=jnp.float32)

        row = lax.broadcasted_iota(jnp.float32, (ROWS, DQ), 0)
        pos = row - SQ * jnp.floor(row / SQ)
        lane = lax.broadcasted_iota(jnp.int32, (ROWS, DQ), 1)
        d = lax.rem(lane, DH)
        pair = (d // 2).astype(jnp.float32)
        freq = jnp.exp(pair * (-jnp.log(10000.0) / (DH // 2)))
        angle = pos * freq
        cos_a = jnp.cos(angle)
        sin_a = jnp.sin(angle)
        is_even = lax.rem(lane, 2) == 0

        def rot(t):
            nxt = jnp.concatenate([t[:, 1:], t[:, :1]], axis=1)
            prv = jnp.concatenate([t[:, -1:], t[:, :-1]], axis=1)
            t_r = jnp.where(is_even, -nxt, prv)
            return t * cos_a + t_r * sin_a

        q = rot(q).astype(jnp.bfloat16)
        k = rot(k).astype(jnp.bfloat16)
        v = v.astype(jnp.bfloat16)

        ctx_rows = []
        for b in range(B):
            heads = []
            for h in range(HQ):
                qh = q[b * SQ:(b + 1) * SQ, h * DH:(h + 1) * DH]
                kh = k[b * SQ:(b + 1) * SQ, h * DH:(h + 1) * DH]
                vh = v[b * SQ:(b + 1) * SQ, h * DH:(h + 1) * DH]
                s = lax.dot_general(
                    qh, kh, (((1,), (1,)), ((), ())),
                    preferred_element_type=jnp.float32,
                ) * 0.125
                m = jnp.max(s, axis=-1, keepdims=True)
                w = jnp.exp(s - m)
                w = w / jnp.sum(w, axis=-1, keepdims=True)
                heads.append(jnp.dot(
                    w.astype(jnp.bfloat16), vh,
                    preferred_element_type=jnp.float32,
                ))
            ctx_rows.append(jnp.concatenate(heads, axis=1))
        ctx = jnp.concatenate(ctx_rows, axis=0)

        partial = jnp.dot(
            ctx.astype(jnp.bfloat16), wo_ref[...].astype(jnp.bfloat16),
            preferred_element_type=jnp.float32,
        )

        comm_ref[0] = partial
        acc = partial

        for h in range(N_DEV - 1):
            rdma = pltpu.make_async_remote_copy(
                src_ref=comm_ref.at[h],
                dst_ref=comm_ref.at[h + 1],
                send_sem=send_sems.at[h],
                recv_sem=recv_sems.at[h + 1],
                device_id=(right,),
                device_id_type=pl.DeviceIdType.MESH,
            )
            rdma.start()
            rdma.wait()
            acc = acc + comm_ref[h + 1]

        out_ref[...] = acc.reshape(B, SQ, D)

    return pl.pallas_call(
        body,
        out_shape=jax.ShapeDtypeStruct((B, SQ, D), jnp.float32),
        in_specs=[pl.BlockSpec(memory_space=pltpu.VMEM)] * 5,
        out_specs=pl.BlockSpec(memory_space=pltpu.VMEM),
        scratch_shapes=[
            pltpu.VMEM((N_DEV, ROWS, D), jnp.float32),
            pltpu.SemaphoreType.DMA((N_DEV,)),
            pltpu.SemaphoreType.DMA((N_DEV,)),
        ],
        compiler_params=pltpu.CompilerParams(collective_id=0),
    )(x, Wq, Wk, Wv, Wo)


# baseline (device time: 35101 ns/iter reference)
import jax
import jax.numpy as jnp
from jax import lax
from jax.experimental import pallas as pl
from jax.experimental.pallas import tpu as pltpu

N_DEV = 4
B, SQ, D = 2, 128, 512
HQ, DH = 4, 64
DQ = HQ * DH
ROWS = B * SQ


def kernel(x, Wq, Wk, Wv, Wo):
    def body(x_ref, wq_ref, wk_ref, wv_ref, wo_ref, out_ref,
             comm_ref, send_sems, recv_sems):
        my = lax.axis_index("i")
        left = lax.rem(my + N_DEV - 1, N_DEV)
        right = lax.rem(my + 1, N_DEV)

        barrier_sem = pltpu.get_barrier_semaphore()
        for nbr in (left, right):
            pl.semaphore_signal(
                barrier_sem, inc=1,
                device_id=(nbr,), device_id_type=pl.DeviceIdType.MESH,
            )
        pl.semaphore_wait(barrier_sem, 2)

        xf = x_ref[...].reshape(ROWS, D).astype(jnp.bfloat16)
        wq = wq_ref[...].astype(jnp.bfloat16)
        wk = wk_ref[...].astype(jnp.bfloat16)
        wv = wv_ref[...].astype(jnp.bfloat16)

        q = jnp.dot(xf, wq, preferred_element_type=jnp.float32)
        k = jnp.dot(xf, wk, preferred_element_type=jnp.float32)
        v = jnp.dot(xf, wv, preferred_element_type=jnp.float32)

        row = lax.broadcasted_iota(jnp.int32, (ROWS, DQ), 0)
        pos = lax.rem(row, SQ).astype(jnp.float32)
        lane = lax.broadcasted_iota(jnp.int32, (ROWS, DQ), 1)
        d = lax.rem(lane, DH)
        pair = (d // 2).astype(jnp.float32)
        freq = jnp.exp(pair * (-jnp.log(10000.0) / (DH // 2)))
        angle = pos * freq
        cos_a = jnp.cos(angle)
        sin_a = jnp.sin(angle)
        is_even = lax.rem(lane, 2) == 0

        def rot(t):
            nxt = jnp.concatenate([t[:, 1:], t[:, :1]], axis=1)
            prv = jnp.concatenate([t[:, -1:], t[:, :-1]], axis=1)
            t_r = jnp.where(is_even, -nxt, prv)
            return t * cos_a + t_r * sin_a

        q = rot(q).astype(jnp.bfloat16)
        k = rot(k).astype(jnp.bfloat16)
        v = v.astype(jnp.bfloat16)

        ctx_rows = []
        for b in range(B):
            heads = []
            for h in range(HQ):
                qh = q[b * SQ:(b + 1) * SQ, h * DH:(h + 1) * DH]
                kh = k[b * SQ:(b + 1) * SQ, h * DH:(h + 1) * DH]
                vh = v[b * SQ:(b + 1) * SQ, h * DH:(h + 1) * DH]
                s = lax.dot_general(
                    qh, kh, (((1,), (1,)), ((), ())),
                    preferred_element_type=jnp.float32,
                ) * 0.125
                m = jnp.max(s, axis=-1, keepdims=True)
                w = jnp.exp(s - m)
                w = w / jnp.sum(w, axis=-1, keepdims=True)
                heads.append(jnp.dot(
                    w.astype(jnp.bfloat16), vh,
                    preferred_element_type=jnp.float32,
                ))
            ctx_rows.append(jnp.concatenate(heads, axis=1))
        ctx = jnp.concatenate(ctx_rows, axis=0)

        partial = jnp.dot(
            ctx.astype(jnp.bfloat16), wo_ref[...].astype(jnp.bfloat16),
            preferred_element_type=jnp.float32,
        )

        comm_ref[0] = partial
        acc = partial

        for h in range(N_DEV - 1):
            rdma = pltpu.make_async_remote_copy(
                src_ref=comm_ref.at[h],
                dst_ref=comm_ref.at[h + 1],
                send_sem=send_sems.at[h],
                recv_sem=recv_sems.at[h + 1],
                device_id=(right,),
                device_id_type=pl.DeviceIdType.MESH,
            )
            rdma.start()
            rdma.wait()
            acc = acc + comm_ref[h + 1]

        out_ref[...] = acc.reshape(B, SQ, D)

    return pl.pallas_call(
        body,
        out_shape=jax.ShapeDtypeStruct((B, SQ, D), jnp.float32),
        in_specs=[pl.BlockSpec(memory_space=pltpu.VMEM)] * 5,
        out_specs=pl.BlockSpec(memory_space=pltpu.VMEM),
        scratch_shapes=[
            pltpu.VMEM((N_DEV, ROWS, D), jnp.float32),
            pltpu.SemaphoreType.DMA((N_DEV,)),
            pltpu.SemaphoreType.DMA((N_DEV,)),
        ],
        compiler_params=pltpu.CompilerParams(collective_id=0),
    )(x, Wq, Wk, Wv, Wo)


# device time: 21765 ns/iter; 1.6127x vs baseline; 1.6127x over previous
import jax
import jax.numpy as jnp
from jax import lax
from jax.experimental import pallas as pl
from jax.experimental.pallas import tpu as pltpu

N_DEV = 4
B, SQ, D = 2, 128, 512
HQ, DH = 4, 64
DQ = HQ * DH
ROWS = B * SQ


def kernel(x, Wq, Wk, Wv, Wo):
    def body(x_ref, wq_ref, wk_ref, wv_ref, wo_ref, out_ref,
             snd_ref, rcv_ref, send_sems, recv_sems):
        my = lax.axis_index("i")
        left = lax.rem(my + N_DEV - 1, N_DEV)
        right = lax.rem(my + 1, N_DEV)
        partner0 = my ^ 1
        partner1 = (N_DEV - 1) - my

        barrier_sem = pltpu.get_barrier_semaphore()
        for nbr in (left, right):
            pl.semaphore_signal(
                barrier_sem, inc=1,
                device_id=(nbr,), device_id_type=pl.DeviceIdType.MESH,
            )
        pl.semaphore_wait(barrier_sem, 2)

        xf = x_ref[...].reshape(ROWS, D).astype(jnp.bfloat16)
        wq = wq_ref[...].astype(jnp.bfloat16)
        wk = wk_ref[...].astype(jnp.bfloat16)
        wv = wv_ref[...].astype(jnp.bfloat16)

        q = jnp.dot(xf, wq, preferred_element_type=jnp.float32)
        k = jnp.dot(xf, wk, preferred_element_type=jnp.float32)
        v = jnp.dot(xf, wv, preferred_element_type=jnp.float32)

        row = lax.broadcasted_iota(jnp.int32, (ROWS, DQ), 0)
        pos = lax.rem(row, SQ).astype(jnp.float32)
        lane = lax.broadcasted_iota(jnp.int32, (ROWS, DQ), 1)
        d = lax.rem(lane, DH)
        pair = (d // 2).astype(jnp.float32)
        freq = jnp.exp(pair * (-jnp.log(10000.0) / (DH // 2)))
        angle = pos * freq
        cos_a = jnp.cos(angle)
        sin_a = jnp.sin(angle)
        is_even = lax.rem(lane, 2) == 0

        def rot(t):
            nxt = jnp.concatenate([t[:, 1:], t[:, :1]], axis=1)
            prv = jnp.concatenate([t[:, -1:], t[:, :-1]], axis=1)
            t_r = jnp.where(is_even, -nxt, prv)
            return t * cos_a + t_r * sin_a

        q = rot(q).astype(jnp.bfloat16)
        k = rot(k).astype(jnp.bfloat16)
        v = v.astype(jnp.bfloat16)

        ctx_rows = []
        for b in range(B):
            heads = []
            for h in range(HQ):
                qh = q[b * SQ:(b + 1) * SQ, h * DH:(h + 1) * DH]
                kh = k[b * SQ:(b + 1) * SQ, h * DH:(h + 1) * DH]
                vh = v[b * SQ:(b + 1) * SQ, h * DH:(h + 1) * DH]
                s = lax.dot_general(
                    qh, kh, (((1,), (1,)), ((), ())),
                    preferred_element_type=jnp.float32,
                ) * 0.125
                m = jnp.max(s, axis=-1, keepdims=True)
                w = jnp.exp(s - m)
                w = w / jnp.sum(w, axis=-1, keepdims=True)
                heads.append(jnp.dot(
                    w.astype(jnp.bfloat16), vh,
                    preferred_element_type=jnp.float32,
                ))
            ctx_rows.append(jnp.concatenate(heads, axis=1))
        ctx = jnp.concatenate(ctx_rows, axis=0)

        partial = jnp.dot(
            ctx.astype(jnp.bfloat16), wo_ref[...].astype(jnp.bfloat16),
            preferred_element_type=jnp.float32,
        )

        acc = partial
        for r, partner in enumerate((partner0, partner1)):
            snd_ref[r] = acc.astype(jnp.bfloat16)
            rdma = pltpu.make_async_remote_copy(
                src_ref=snd_ref.at[r],
                dst_ref=rcv_ref.at[r],
                send_sem=send_sems.at[r],
                recv_sem=recv_sems.at[r],
                device_id=(partner,),
                device_id_type=pl.DeviceIdType.MESH,
            )
            rdma.start()
            rdma.wait()
            acc = acc + rcv_ref[r].astype(jnp.float32)

        out_ref[...] = acc.reshape(B, SQ, D)

    return pl.pallas_call(
        body,
        out_shape=jax.ShapeDtypeStruct((B, SQ, D), jnp.float32),
        in_specs=[pl.BlockSpec(memory_space=pltpu.VMEM)] * 5,
        out_specs=pl.BlockSpec(memory_space=pltpu.VMEM),
        scratch_shapes=[
            pltpu.VMEM((2, ROWS, D), jnp.bfloat16),
            pltpu.VMEM((2, ROWS, D), jnp.bfloat16),
            pltpu.SemaphoreType.DMA((2,)),
            pltpu.SemaphoreType.DMA((2,)),
        ],
        compiler_params=pltpu.CompilerParams(collective_id=0),
    )(x, Wq, Wk, Wv, Wo)


# device time: 19130 ns/iter; 1.8349x vs baseline; 1.1377x over previous
import jax
import jax.numpy as jnp
from jax import lax
from jax.experimental import pallas as pl
from jax.experimental.pallas import tpu as pltpu

N_DEV = 4
B, SQ, D = 2, 128, 512
HQ, DH = 4, 64
DQ = HQ * DH


def kernel(x, Wq, Wk, Wv, Wo):
    def body(x_ref, wq_ref, wk_ref, wv_ref, wo_ref, out_ref,
             snd_ref, rcv_ref, send_sems, recv_sems):
        my = lax.axis_index("i")
        left = lax.rem(my + N_DEV - 1, N_DEV)
        right = lax.rem(my + 1, N_DEV)
        partners = (my ^ 1, (N_DEV - 1) - my)

        pos = lax.broadcasted_iota(jnp.int32, (SQ, DQ), 0).astype(jnp.float32)
        lane = lax.broadcasted_iota(jnp.int32, (SQ, DQ), 1)
        d = lax.rem(lane, DH)
        pair = (d // 2).astype(jnp.float32)
        freq = jnp.exp(pair * (-jnp.log(10000.0) / (DH // 2)))
        angle = pos * freq
        cos_a = jnp.cos(angle)
        sin_a = jnp.sin(angle)
        is_even = lax.rem(lane, 2) == 0

        def rot(t):
            nxt = jnp.concatenate([t[:, 1:], t[:, :1]], axis=1)
            prv = jnp.concatenate([t[:, -1:], t[:, :-1]], axis=1)
            t_r = jnp.where(is_even, -nxt, prv)
            return t * cos_a + t_r * sin_a

        wq = wq_ref[...].astype(jnp.bfloat16)
        wk = wk_ref[...].astype(jnp.bfloat16)
        wv = wv_ref[...].astype(jnp.bfloat16)
        wo = wo_ref[...].astype(jnp.bfloat16)

        def compute_partial(b):
            xb = x_ref[b].astype(jnp.bfloat16)
            q = jnp.dot(xb, wq, preferred_element_type=jnp.float32)
            k = jnp.dot(xb, wk, preferred_element_type=jnp.float32)
            v = jnp.dot(xb, wv,
                        preferred_element_type=jnp.float32).astype(jnp.bfloat16)
            q = rot(q).astype(jnp.bfloat16)
            k = rot(k).astype(jnp.bfloat16)
            heads = []
            for h in range(HQ):
                qh = q[:, h * DH:(h + 1) * DH]
                kh = k[:, h * DH:(h + 1) * DH]
                vh = v[:, h * DH:(h + 1) * DH]
                s = lax.dot_general(
                    qh, kh, (((1,), (1,)), ((), ())),
                    preferred_element_type=jnp.float32,
                ) * 0.125
                m = jnp.max(s, axis=-1, keepdims=True)
                w = jnp.exp(s - m)
                w = w / jnp.sum(w, axis=-1, keepdims=True)
                heads.append(jnp.dot(
                    w.astype(jnp.bfloat16), vh,
                    preferred_element_type=jnp.float32,
                ))
            ctx = jnp.concatenate(heads, axis=1)
            return jnp.dot(ctx.astype(jnp.bfloat16), wo,
                           preferred_element_type=jnp.float32)

        def exchange(r, b, data_bf16):
            snd_ref[r, b] = data_bf16
            rdma = pltpu.make_async_remote_copy(
                src_ref=snd_ref.at[r, b],
                dst_ref=rcv_ref.at[r, b],
                send_sem=send_sems.at[r, b],
                recv_sem=recv_sems.at[r, b],
                device_id=(partners[r],),
                device_id_type=pl.DeviceIdType.MESH,
            )
            rdma.start()
            return rdma

        p0 = compute_partial(0)

        barrier_sem = pltpu.get_barrier_semaphore()
        for nbr in (left, right):
            pl.semaphore_signal(
                barrier_sem, inc=1,
                device_id=(nbr,), device_id_type=pl.DeviceIdType.MESH,
            )
        pl.semaphore_wait(barrier_sem, 2)

        r00 = exchange(0, 0, p0.astype(jnp.bfloat16))
        p1 = compute_partial(1)
        r01 = exchange(0, 1, p1.astype(jnp.bfloat16))
        r00.wait()
        a0 = p0 + rcv_ref[0, 0].astype(jnp.float32)
        r10 = exchange(1, 0, a0.astype(jnp.bfloat16))
        r01.wait()
        a1 = p1 + rcv_ref[0, 1].astype(jnp.float32)
        r11 = exchange(1, 1, a1.astype(jnp.bfloat16))
        r10.wait()
        out_ref[0] = a0 + rcv_ref[1, 0].astype(jnp.float32)
        r11.wait()
        out_ref[1] = a1 + rcv_ref[1, 1].astype(jnp.float32)

    return pl.pallas_call(
        body,
        out_shape=jax.ShapeDtypeStruct((B, SQ, D), jnp.float32),
        in_specs=[pl.BlockSpec(memory_space=pltpu.VMEM)] * 5,
        out_specs=pl.BlockSpec(memory_space=pltpu.VMEM),
        scratch_shapes=[
            pltpu.VMEM((2, 2, SQ, D), jnp.bfloat16),
            pltpu.VMEM((2, 2, SQ, D), jnp.bfloat16),
            pltpu.SemaphoreType.DMA((2, 2)),
            pltpu.SemaphoreType.DMA((2, 2)),
        ],
        compiler_params=pltpu.CompilerParams(collective_id=0),
    )(x, Wq, Wk, Wv, Wo)


# device time: 19116 ns/iter; 1.8362x vs baseline; 1.0007x over previous
import jax
import jax.numpy as jnp
from jax import lax
from jax.experimental import pallas as pl
from jax.experimental.pallas import tpu as pltpu

N_DEV = 4
B, SQ, D = 2, 128, 512
HQ, DH = 4, 64
DQ = HQ * DH


def kernel(x, Wq, Wk, Wv, Wo):
    def body(x_ref, wq_ref, wk_ref, wv_ref, wo_ref, out_ref,
             snd_ref, rcv_ref, send_sems, recv_sems):
        my = lax.axis_index("i")
        left = lax.rem(my + N_DEV - 1, N_DEV)
        right = lax.rem(my + 1, N_DEV)
        diag = lax.rem(my + 2, N_DEV)

        pos = lax.broadcasted_iota(jnp.int32, (SQ, DQ), 0).astype(jnp.float32)
        lane = lax.broadcasted_iota(jnp.int32, (SQ, DQ), 1)
        d = lax.rem(lane, DH)
        pair = (d // 2).astype(jnp.float32)
        freq = jnp.exp(pair * (-jnp.log(10000.0) / (DH // 2)))
        angle = pos * freq
        cos_a = jnp.cos(angle)
        sin_a = jnp.sin(angle)
        is_even = lax.rem(lane, 2) == 0

        def rot(t):
            nxt = jnp.concatenate([t[:, 1:], t[:, :1]], axis=1)
            prv = jnp.concatenate([t[:, -1:], t[:, :-1]], axis=1)
            t_r = jnp.where(is_even, -nxt, prv)
            return t * cos_a + t_r * sin_a

        wq = wq_ref[...].astype(jnp.bfloat16)
        wk = wk_ref[...].astype(jnp.bfloat16)
        wv = wv_ref[...].astype(jnp.bfloat16)
        wo = wo_ref[...].astype(jnp.bfloat16)

        def compute_partial(b):
            xb = x_ref[b].astype(jnp.bfloat16)
            q = jnp.dot(xb, wq, preferred_element_type=jnp.float32)
            k = jnp.dot(xb, wk, preferred_element_type=jnp.float32)
            v = jnp.dot(xb, wv,
                        preferred_element_type=jnp.float32).astype(jnp.bfloat16)
            q = rot(q).astype(jnp.bfloat16)
            k = rot(k).astype(jnp.bfloat16)
            heads = []
            for h in range(HQ):
                qh = q[:, h * DH:(h + 1) * DH]
                kh = k[:, h * DH:(h + 1) * DH]
                vh = v[:, h * DH:(h + 1) * DH]
                s = lax.dot_general(
                    qh, kh, (((1,), (1,)), ((), ())),
                    preferred_element_type=jnp.float32,
                ) * 0.125
                m = jnp.max(s, axis=-1, keepdims=True)
                w = jnp.exp(s - m)
                w = w / jnp.sum(w, axis=-1, keepdims=True)
                heads.append(jnp.dot(
                    w.astype(jnp.bfloat16), vh,
                    preferred_element_type=jnp.float32,
                ))
            ctx = jnp.concatenate(heads, axis=1)
            return jnp.dot(ctx.astype(jnp.bfloat16), wo,
                           preferred_element_type=jnp.float32)

        targets = ((right, 0), (left, 1), (diag, 2))

        def broadcast_half(b, p_bf16):
            snd_ref[b] = p_bf16
            descs = []
            for tgt, slot in targets:
                rdma = pltpu.make_async_remote_copy(
                    src_ref=snd_ref.at[b],
                    dst_ref=rcv_ref.at[slot, b],
                    send_sem=send_sems.at[slot, b],
                    recv_sem=recv_sems.at[slot, b],
                    device_id=(tgt,),
                    device_id_type=pl.DeviceIdType.MESH,
                )
                rdma.start()
                descs.append(rdma)
            return descs

        def recv_half(b):
            for slot in range(3):
                rdma = pltpu.make_async_remote_copy(
                    src_ref=snd_ref.at[b],
                    dst_ref=rcv_ref.at[slot, b],
                    send_sem=send_sems.at[slot, b],
                    recv_sem=recv_sems.at[slot, b],
                    device_id=(right,),
                    device_id_type=pl.DeviceIdType.MESH,
                )
                rdma.wait_recv()

        p0 = compute_partial(0)

        barrier_sem = pltpu.get_barrier_semaphore()
        for nbr in (left, right, diag):
            pl.semaphore_signal(
                barrier_sem, inc=1,
                device_id=(nbr,), device_id_type=pl.DeviceIdType.MESH,
            )
        pl.semaphore_wait(barrier_sem, 3)

        s0 = broadcast_half(0, p0.astype(jnp.bfloat16))
        p1 = compute_partial(1)
        s1 = broadcast_half(1, p1.astype(jnp.bfloat16))

        recv_half(0)
        out_ref[0] = (p0
                      + rcv_ref[0, 0].astype(jnp.float32)
                      + rcv_ref[1, 0].astype(jnp.float32)
                      + rcv_ref[2, 0].astype(jnp.float32))
        recv_half(1)
        out_ref[1] = (p1
                      + rcv_ref[0, 1].astype(jnp.float32)
                      + rcv_ref[1, 1].astype(jnp.float32)
                      + rcv_ref[2, 1].astype(jnp.float32))

        for rdma in s0 + s1:
            rdma.wait_send()

    return pl.pallas_call(
        body,
        out_shape=jax.ShapeDtypeStruct((B, SQ, D), jnp.float32),
        in_specs=[pl.BlockSpec(memory_space=pltpu.VMEM)] * 5,
        out_specs=pl.BlockSpec(memory_space=pltpu.VMEM),
        scratch_shapes=[
            pltpu.VMEM((2, SQ, D), jnp.bfloat16),
            pltpu.VMEM((3, 2, SQ, D), jnp.bfloat16),
            pltpu.SemaphoreType.DMA((3, 2)),
            pltpu.SemaphoreType.DMA((3, 2)),
        ],
        compiler_params=pltpu.CompilerParams(collective_id=0),
    )(x, Wq, Wk, Wv, Wo)


# device time: 17494 ns/iter; 2.0065x vs baseline; 1.0927x over previous
import jax
import jax.numpy as jnp
from jax import lax
from jax.experimental import pallas as pl
from jax.experimental.pallas import tpu as pltpu

N_DEV = 4
B, SQ, D = 2, 128, 512
HQ, DH = 4, 64
DQ = HQ * DH
NC = 4
CR = B * SQ // NC


def kernel(x, Wq, Wk, Wv, Wo):
    def body(x_ref, wq_ref, wk_ref, wv_ref, wo_ref, out_ref,
             snd_ref, rcv_ref, send_sems, recv_sems):
        my = lax.axis_index("i")
        left = lax.rem(my + N_DEV - 1, N_DEV)
        right = lax.rem(my + 1, N_DEV)
        partner_a = my ^ 1
        partner_b = (N_DEV - 1) - my

        barrier_sem = pltpu.get_barrier_semaphore()
        for nbr in (left, right):
            pl.semaphore_signal(
                barrier_sem, inc=1,
                device_id=(nbr,), device_id_type=pl.DeviceIdType.MESH,
            )

        pos = lax.broadcasted_iota(jnp.int32, (SQ, DQ), 0).astype(jnp.float32)
        lane = lax.broadcasted_iota(jnp.int32, (SQ, DQ), 1)
        d = lax.rem(lane, DH)
        pair = (d // 2).astype(jnp.float32)
        freq = jnp.exp(pair * (-jnp.log(10000.0) / (DH // 2)))
        angle = pos * freq
        cos_a = jnp.cos(angle)
        sin_a = jnp.sin(angle)
        is_even = lax.rem(lane, 2) == 0

        def rot(t):
            nxt = jnp.concatenate([t[:, 1:], t[:, :1]], axis=1)
            prv = jnp.concatenate([t[:, -1:], t[:, :-1]], axis=1)
            t_r = jnp.where(is_even, -nxt, prv)
            return t * cos_a + t_r * sin_a

        wq = wq_ref[...].astype(jnp.bfloat16)
        wk = wk_ref[...].astype(jnp.bfloat16)
        wv = wv_ref[...].astype(jnp.bfloat16)
        wo = wo_ref[...].astype(jnp.bfloat16)

        def compute_partial(b):
            xb = x_ref[b].astype(jnp.bfloat16)
            q = jnp.dot(xb, wq, preferred_element_type=jnp.float32)
            k = jnp.dot(xb, wk, preferred_element_type=jnp.float32)
            v = jnp.dot(xb, wv,
                        preferred_element_type=jnp.float32).astype(jnp.bfloat16)
            q = rot(q).astype(jnp.bfloat16)
            k = rot(k).astype(jnp.bfloat16)
            heads = []
            for h in range(HQ):
                qh = q[:, h * DH:(h + 1) * DH]
                kh = k[:, h * DH:(h + 1) * DH]
                vh = v[:, h * DH:(h + 1) * DH]
                s = lax.dot_general(
                    qh, kh, (((1,), (1,)), ((), ())),
                    preferred_element_type=jnp.float32,
                ) * 0.125
                m = jnp.max(s, axis=-1, keepdims=True)
                w = jnp.exp(s - m)
                w = w / jnp.sum(w, axis=-1, keepdims=True)
                heads.append(jnp.dot(
                    w.astype(jnp.bfloat16), vh,
                    preferred_element_type=jnp.float32,
                ))
            ctx = jnp.concatenate(heads, axis=1)
            return jnp.dot(ctx.astype(jnp.bfloat16), wo,
                           preferred_element_type=jnp.float32)

        def chunk_partner(c, r):
            return (partner_a, partner_b)[(c + r) % 2]

        def exchange(r, c, data_bf16):
            snd_ref[r, c] = data_bf16
            rdma = pltpu.make_async_remote_copy(
                src_ref=snd_ref.at[r, c],
                dst_ref=rcv_ref.at[r, c],
                send_sem=send_sems.at[r, c],
                recv_sem=recv_sems.at[r, c],
                device_id=(chunk_partner(c, r),),
                device_id_type=pl.DeviceIdType.MESH,
            )
            rdma.start()
            return rdma

        p0 = compute_partial(0)
        pl.semaphore_wait(barrier_sem, 2)

        r0 = [None] * NC
        r1 = [None] * NC
        acc = [None] * NC
        r0[0] = exchange(0, 0, p0[:CR].astype(jnp.bfloat16))
        r0[1] = exchange(0, 1, p0[CR:].astype(jnp.bfloat16))

        p1 = compute_partial(1)
        r0[2] = exchange(0, 2, p1[:CR].astype(jnp.bfloat16))
        r0[3] = exchange(0, 3, p1[CR:].astype(jnp.bfloat16))

        parts = (p0[:CR], p0[CR:], p1[:CR], p1[CR:])
        for c in range(NC):
            r0[c].wait()
            acc[c] = parts[c] + rcv_ref[0, c].astype(jnp.float32)
            r1[c] = exchange(1, c, acc[c].astype(jnp.bfloat16))
        for c in range(NC):
            r1[c].wait()
            b, half = divmod(c, 2)
            out_ref[b, half * CR:(half + 1) * CR, :] = (
                acc[c] + rcv_ref[1, c].astype(jnp.float32)
            ).astype(out_ref.dtype)

    return pl.pallas_call(
        body,
        out_shape=jax.ShapeDtypeStruct((B, SQ, D), jnp.float32),
        in_specs=[pl.BlockSpec(memory_space=pltpu.VMEM)] * 5,
        out_specs=pl.BlockSpec(memory_space=pltpu.VMEM),
        scratch_shapes=[
            pltpu.VMEM((2, NC, CR, D), jnp.bfloat16),
            pltpu.VMEM((2, NC, CR, D), jnp.bfloat16),
            pltpu.SemaphoreType.DMA((2, NC)),
            pltpu.SemaphoreType.DMA((2, NC)),
        ],
        compiler_params=pltpu.CompilerParams(collective_id=0),
    )(x, Wq, Wk, Wv, Wo)
